# baseline (speedup 1.0000x reference)
import functools

import jax
import jax.numpy as jnp
from jax import lax
from jax.experimental import pallas as pl
from jax.experimental.pallas import tpu as pltpu
from jax.experimental.pallas import tpu_sc as plsc

N = 10000
E = 320000
D = 128
NC = 2
NS = 16
NW = NC * NS
CH = 128
KTOT = 160
DK0 = 128
DK1 = KTOT - DK0
AK0 = 72
AK1 = KTOT // 2 - AK0
EP = NS * KTOT * CH
ACC_ROWS = 10112
STRIPE = ACC_ROWS // NS
NBUF = 2

_mesh = plsc.VectorSubcoreMesh(core_axis_name="c", subcore_axis_name="s")


def _unpack_chunk(packed, j, rowchunk, colchunk):
    for v in range(CH // 16):
        p = packed[j, pl.ds(v * 16, 16)]
        rowchunk[pl.ds(v * 16, 16)] = lax.bitwise_and(p, 0xFFFF)
        colchunk[pl.ds(v * 16, 16)] = lax.shift_right_logical(p, 16)


def _zero_vmem_2d(buf, rows, cols):
    def body(i, _):
        for j in range(cols // 16):
            buf[i, pl.ds(j * 16, 16)] = jnp.zeros((16,), jnp.float32)
        return 0
    lax.fori_loop(0, rows, body, 0)


def _zero_vmem_1d(buf, n):
    def body(i, _):
        buf[pl.ds(i * 16, 16)] = jnp.zeros((16,), jnp.float32)
        return 0
    lax.fori_loop(0, n // 16, body, 0)


def _load_edges_split(edges_hbm, ebuf, c, s, k0, k1):
    @pl.when(c == 0)
    def _():
        pltpu.sync_copy(edges_hbm.at[pl.ds(s * k0, k0)], ebuf.at[pl.ds(0, k0)])

    @pl.when(c == 1)
    def _():
        pltpu.sync_copy(edges_hbm.at[pl.ds(NS * k0 + s * k1, k1)],
                        ebuf.at[pl.ds(0, k1)])


@functools.partial(
    pl.kernel,
    out_type=jax.ShapeDtypeStruct((NC * ACC_ROWS,), jnp.float32),
    mesh=_mesh,
    scratch_types=[
        pltpu.VMEM((DK0, CH), jnp.int32),
        pltpu.VMEM((CH,), jnp.int32),
        pltpu.VMEM((CH,), jnp.int32),
        pltpu.VMEM((CH,), jnp.float32),
        pltpu.VMEM((CH,), jnp.float32),
        pltpu.VMEM((STRIPE,), jnp.float32),
        pltpu.VMEM_SHARED((ACC_ROWS,), jnp.float32),
    ],
)
def _deg_kernel(edges_hbm, out_hbm, ebuf, colchunk, rowchunk, ones, zbuf, stage, acc):
    c = lax.axis_index("c")
    s = lax.axis_index("s")
    _load_edges_split(edges_hbm, ebuf, c, s, DK0, DK1)
    for j in range(CH // 16):
        ones[pl.ds(j * 16, 16)] = jnp.ones((16,), jnp.float32)
    _zero_vmem_1d(zbuf, CH)
    base = s * STRIPE
    for k in range(STRIPE // CH):
        pltpu.sync_copy(zbuf, acc.at[pl.ds(base + k * CH, CH)])
    pltpu.sync_copy(zbuf.at[pl.ds(0, STRIPE % CH)],
                    acc.at[pl.ds(base + (STRIPE // CH) * CH, STRIPE % CH)])
    plsc.subcore_barrier()
    nchunks = jnp.where(c == 0, DK0, DK1)

    def body(j, _):
        _unpack_chunk(ebuf, j, rowchunk, colchunk)
        pltpu.sync_copy(ones, acc.at[colchunk], add=True)
        return 0

    lax.fori_loop(0, nchunks, body, 0)
    plsc.subcore_barrier()
    pltpu.sync_copy(acc.at[pl.ds(base, STRIPE)], stage)
    pltpu.sync_copy(stage, out_hbm.at[pl.ds(c * ACC_ROWS + base, STRIPE)])


@functools.partial(
    pl.kernel,
    out_type=jax.ShapeDtypeStruct((NC, ACC_ROWS, D), jnp.float32),
    mesh=_mesh,
    scratch_types=[
        pltpu.VMEM((AK0, CH), jnp.int32),
        pltpu.VMEM((NBUF, CH), jnp.int32),
        pltpu.VMEM((NBUF, CH), jnp.int32),
        pltpu.VMEM((NBUF, CH, D), jnp.float32),
        pltpu.VMEM_SHARED((ACC_ROWS, D), jnp.float32),
        pltpu.SemaphoreType.DMA,
        pltpu.SemaphoreType.DMA,
    ],
)
def _agg_kernel(edges_hbm, y_hbm, out_hbm, ebuf, rowc, colc, mbuf, acc, sem0, sem1):
    sems = (sem0, sem1)
    c = lax.axis_index("c")
    s = lax.axis_index("s")
    _load_edges_split(edges_hbm, ebuf, c, s, AK0, AK1)
    _zero_vmem_2d(mbuf.at[0], CH, D)
    base = s * STRIPE
    for k in range(STRIPE // CH):
        pltpu.sync_copy(mbuf.at[0], acc.at[pl.ds(base + k * CH, CH)])
    pltpu.sync_copy(mbuf.at[0].at[pl.ds(0, STRIPE % CH)],
                    acc.at[pl.ds(base + (STRIPE // CH) * CH, STRIPE % CH)])
    plsc.subcore_barrier()
    nchunks = jnp.where(c == 0, AK0, AK1)

    for b in range(NBUF):
        _unpack_chunk(ebuf, b, rowc.at[b], colc.at[b])
        pltpu.async_copy(y_hbm.at[rowc.at[b]], mbuf.at[b], sems[b])

    def body(i, _):
        for b in range(NBUF):
            j = i * NBUF + b
            pltpu.make_async_copy(y_hbm.at[rowc.at[b]], mbuf.at[b], sems[b]).wait()
            pltpu.sync_copy(mbuf.at[b], acc.at[colc.at[b]], add=True)
            jn = jnp.minimum(j + NBUF, nchunks - 1)
            _unpack_chunk(ebuf, jn, rowc.at[b], colc.at[b])
            pltpu.async_copy(y_hbm.at[rowc.at[b]], mbuf.at[b], sems[b])
        return 0

    lax.fori_loop(0, nchunks // NBUF, body, 0)
    for b in range(NBUF):
        pltpu.make_async_copy(y_hbm.at[rowc.at[b]], mbuf.at[b], sems[b]).wait()
    plsc.subcore_barrier()
    pltpu.sync_copy(acc.at[pl.ds(base, STRIPE)], out_hbm.at[c, pl.ds(base, STRIPE)])


@functools.partial(
    pl.kernel,
    out_type=jax.ShapeDtypeStruct((NC * ACC_ROWS,), jnp.float32),
    mesh=_mesh,
    scratch_types=[
        pltpu.VMEM((DK0, CH), jnp.int32),
        pltpu.VMEM((NBUF, CH), jnp.int32),
        pltpu.VMEM((NBUF, CH), jnp.int32),
        pltpu.VMEM((NBUF, CH), jnp.float32),
        pltpu.VMEM((STRIPE,), jnp.float32),
        pltpu.VMEM_SHARED((ACC_ROWS,), jnp.float32),
        pltpu.SemaphoreType.DMA,
        pltpu.SemaphoreType.DMA,
    ],
)
def _agg1_kernel(edges_hbm, y_hbm, out_hbm, ebuf, rowc, colc, mbuf, stage, acc,
                 sem0, sem1):
    sems = (sem0, sem1)
    c = lax.axis_index("c")
    s = lax.axis_index("s")
    _load_edges_split(edges_hbm, ebuf, c, s, DK0, DK1)
    _zero_vmem_1d(mbuf.at[0], CH)
    base = s * STRIPE
    for k in range(STRIPE // CH):
        pltpu.sync_copy(mbuf.at[0], acc.at[pl.ds(base + k * CH, CH)])
    pltpu.sync_copy(mbuf.at[0].at[pl.ds(0, STRIPE % CH)],
                    acc.at[pl.ds(base + (STRIPE // CH) * CH, STRIPE % CH)])
    plsc.subcore_barrier()
    nchunks = jnp.where(c == 0, DK0, DK1)

    for b in range(NBUF):
        _unpack_chunk(ebuf, b, rowc.at[b], colc.at[b])
        pltpu.async_copy(y_hbm.at[rowc.at[b]], mbuf.at[b], sems[b])

    def body(i, _):
        for b in range(NBUF):
            j = i * NBUF + b
            pltpu.make_async_copy(y_hbm.at[rowc.at[b]], mbuf.at[b], sems[b]).wait()
            pltpu.sync_copy(mbuf.at[b], acc.at[colc.at[b]], add=True)
            jn = jnp.minimum(j + NBUF, nchunks - 1)
            _unpack_chunk(ebuf, jn, rowc.at[b], colc.at[b])
            pltpu.async_copy(y_hbm.at[rowc.at[b]], mbuf.at[b], sems[b])
        return 0

    lax.fori_loop(0, nchunks // NBUF, body, 0)
    for b in range(NBUF):
        pltpu.make_async_copy(y_hbm.at[rowc.at[b]], mbuf.at[b], sems[b]).wait()
    plsc.subcore_barrier()
    pltpu.sync_copy(acc.at[pl.ds(base, STRIPE)], stage)
    pltpu.sync_copy(stage, out_hbm.at[pl.ds(c * ACC_ROWS + base, STRIPE)])


RB = 1000
GRID = N // RB


def _stage_in_body(x_ref, w_ref, cnt0_ref, cnt1_ref, xw_ref, y_ref, dis2_ref, dis1_ref):
    xw = jnp.dot(x_ref[...], w_ref[...], preferred_element_type=jnp.float32)
    deg = cnt0_ref[...] + cnt1_ref[...]
    dis2 = lax.rsqrt(deg + 2.0)
    dis1 = lax.rsqrt(deg + 1.0)
    xw_ref[...] = xw
    y_ref[...] = dis2 * xw
    dis2_ref[...] = dis2
    dis1_ref[...] = dis1


def _stage_mid_body(agga_ref, aggb_ref, xw_ref, dis2_ref, b_ref, w_ref,
                    yprev_ref, xw2_ref, y2_ref):
    del yprev_ref
    dis2 = dis2_ref[...]
    agg = agga_ref[0] + agga_ref[1] + aggb_ref[0] + aggb_ref[1]
    h = dis2 * agg + dis2 * dis2 * xw_ref[...] + b_ref[...]
    h = jnp.maximum(h, 0.0)
    xw2 = jnp.dot(h, w_ref[...], preferred_element_type=jnp.float32)
    xw2_ref[...] = xw2
    y2_ref[...] = dis2 * xw2


def _stage_out_body(agga_ref, aggb_ref, xw_ref, dis2_ref, dis1_ref, b_ref,
                    wo_ref, xwo_ref, yo_ref):
    dis2 = dis2_ref[...]
    agg = agga_ref[0] + agga_ref[1] + aggb_ref[0] + aggb_ref[1]
    h = dis2 * agg + dis2 * dis2 * xw_ref[...] + b_ref[...]
    h = jnp.maximum(h, 0.0)
    xwo = jnp.dot(h, wo_ref[...], preferred_element_type=jnp.float32)
    xwo_ref[...] = xwo
    yo_ref[...] = dis1_ref[...] * xwo


def _stage_final_body(aggo_ref, xwo_ref, dis1_ref, bo_ref, out_ref):
    dis1 = dis1_ref[...]
    out_ref[...] = dis1 * (aggo_ref[0] + aggo_ref[1]) + dis1 * dis1 * xwo_ref[...] + bo_ref[...]


def _rowspec(width):
    return pl.BlockSpec((RB, width), lambda i: (i, 0))


def _aggspec(width):
    return pl.BlockSpec((2, RB, width), lambda i: (0, i, 0))


def _fullspec(shape):
    return pl.BlockSpec(shape, lambda i: tuple(0 for _ in shape))


def kernel(x, edge_index, W1, b1, W2, b2, W3, b3, Wo, bo):
    row = edge_index[0]
    col = edge_index[1]
    pad = EP - E
    row_p = jnp.concatenate([row, jnp.zeros((pad,), jnp.int32)])
    col_p = jnp.concatenate([col, jnp.full((pad,), N, jnp.int32)])
    edges_p = (row_p | (col_p << 16)).reshape(NS * KTOT, CH)

    cnt = _deg_kernel(edges_p).reshape(NC, ACC_ROWS)
    cnt0 = cnt[0][:, None]
    cnt1 = cnt[1][:, None]

    stage_in = pl.pallas_call(
        _stage_in_body,
        grid=(GRID,),
        in_specs=[_rowspec(D), _fullspec((D, D)), _rowspec(1), _rowspec(1)],
        out_specs=[_rowspec(D), _rowspec(D), _rowspec(1), _rowspec(1)],
        out_shape=[
            jax.ShapeDtypeStruct((N, D), jnp.float32),
            jax.ShapeDtypeStruct((N, D), jnp.float32),
            jax.ShapeDtypeStruct((N, 1), jnp.float32),
            jax.ShapeDtypeStruct((N, 1), jnp.float32),
        ],
    )
    xw1, y1, dis2, dis1 = stage_in(x, W1, cnt0, cnt1)

    stage_mid = pl.pallas_call(
        _stage_mid_body,
        grid=(GRID,),
        in_specs=[_aggspec(D), _aggspec(D), _rowspec(D), _rowspec(1),
                  _fullspec((1, D)), _fullspec((D, D)), _rowspec(D)],
        out_specs=[_rowspec(D), _rowspec(D)],
        out_shape=[
            jax.ShapeDtypeStruct((N, D), jnp.float32),
            jax.ShapeDtypeStruct((N, D), jnp.float32),
        ],
        input_output_aliases={6: 1},
    )

    e_a = edges_p[:NS * (KTOT // 2)]
    e_b = edges_p[NS * (KTOT // 2):]

    agg1a = _agg_kernel(e_a, y1)
    agg1b = _agg_kernel(e_b, y1)
    xw2, y2 = stage_mid(agg1a, agg1b, xw1, dis2, b1[None, :], W2, y1)

    agg2a = _agg_kernel(e_a, y2)
    agg2b = _agg_kernel(e_b, y2)
    xw3, y3 = stage_mid(agg2a, agg2b, xw2, dis2, b2[None, :], W3, y2)

    agg3a = _agg_kernel(e_a, y3)
    agg3b = _agg_kernel(e_b, y3)
    stage_out = pl.pallas_call(
        _stage_out_body,
        grid=(GRID,),
        in_specs=[_aggspec(D), _aggspec(D), _rowspec(D), _rowspec(1),
                  _rowspec(1), _fullspec((1, D)), _fullspec((D, 1))],
        out_specs=[_rowspec(1), _rowspec(1)],
        out_shape=[
            jax.ShapeDtypeStruct((N, 1), jnp.float32),
            jax.ShapeDtypeStruct((N, 1), jnp.float32),
        ],
    )
    xwo, yo = stage_out(agg3a, agg3b, xw3, dis2, dis1, b3[None, :], Wo)

    aggo = _agg1_kernel(edges_p, yo.reshape(N)).reshape(NC, ACC_ROWS)

    stage_final = pl.pallas_call(
        _stage_final_body,
        grid=(GRID,),
        in_specs=[_aggspec(1), _rowspec(1), _rowspec(1), _fullspec((1, 1))],
        out_specs=_rowspec(1),
        out_shape=jax.ShapeDtypeStruct((N, 1), jnp.float32),
    )
    out = stage_final(aggo[:, :, None], xwo, dis1, bo[None, :])
    return out


# speedup vs baseline: 1.0877x; 1.0877x over previous
import functools

import jax
import jax.numpy as jnp
from jax import lax
from jax.experimental import pallas as pl
from jax.experimental.pallas import tpu as pltpu
from jax.experimental.pallas import tpu_sc as plsc

N = 10000
E = 320000
D = 128
NC = 2
NS = 16
NW = NC * NS
CH = 128
KTOT = 160
DK0 = 128
DK1 = KTOT - DK0
AK0 = 64
AK1 = KTOT // 2 - AK0
EP = NS * KTOT * CH
ACC_ROWS = 10112
STRIPE = ACC_ROWS // NS
NBUF = 2

_mesh = plsc.VectorSubcoreMesh(core_axis_name="c", subcore_axis_name="s")


def _unpack_chunk(packed, j, rowchunk, colchunk):
    for v in range(CH // 16):
        p = packed[j, pl.ds(v * 16, 16)]
        rowchunk[pl.ds(v * 16, 16)] = lax.bitwise_and(p, 0xFFFF)
        colchunk[pl.ds(v * 16, 16)] = lax.shift_right_logical(p, 16)


def _zero_vmem_2d(buf, rows, cols):
    def body(i, _):
        for j in range(cols // 16):
            buf[i, pl.ds(j * 16, 16)] = jnp.zeros((16,), jnp.float32)
        return 0
    lax.fori_loop(0, rows, body, 0)


def _zero_vmem_1d(buf, n):
    def body(i, _):
        buf[pl.ds(i * 16, 16)] = jnp.zeros((16,), jnp.float32)
        return 0
    lax.fori_loop(0, n // 16, body, 0)


def _load_edges_split(edges_hbm, ebuf, c, s, k0, k1):
    @pl.when(c == 0)
    def _():
        pltpu.sync_copy(edges_hbm.at[pl.ds(s * k0, k0)], ebuf.at[pl.ds(0, k0)])

    @pl.when(c == 1)
    def _():
        pltpu.sync_copy(edges_hbm.at[pl.ds(NS * k0 + s * k1, k1)],
                        ebuf.at[pl.ds(0, k1)])


@functools.partial(
    pl.kernel,
    out_type=jax.ShapeDtypeStruct((NC * ACC_ROWS,), jnp.float32),
    mesh=_mesh,
    scratch_types=[
        pltpu.VMEM((DK0, CH), jnp.int32),
        pltpu.VMEM((CH,), jnp.int32),
        pltpu.VMEM((CH,), jnp.int32),
        pltpu.VMEM((CH,), jnp.float32),
        pltpu.VMEM((CH,), jnp.float32),
        pltpu.VMEM((STRIPE,), jnp.float32),
        pltpu.VMEM_SHARED((ACC_ROWS,), jnp.float32),
    ],
)
def _deg_kernel(edges_hbm, out_hbm, ebuf, colchunk, rowchunk, ones, zbuf, stage, acc):
    c = lax.axis_index("c")
    s = lax.axis_index("s")
    _load_edges_split(edges_hbm, ebuf, c, s, DK0, DK1)
    for j in range(CH // 16):
        ones[pl.ds(j * 16, 16)] = jnp.ones((16,), jnp.float32)
    _zero_vmem_1d(zbuf, CH)
    base = s * STRIPE
    for k in range(STRIPE // CH):
        pltpu.sync_copy(zbuf, acc.at[pl.ds(base + k * CH, CH)])
    pltpu.sync_copy(zbuf.at[pl.ds(0, STRIPE % CH)],
                    acc.at[pl.ds(base + (STRIPE // CH) * CH, STRIPE % CH)])
    plsc.subcore_barrier()
    nchunks = jnp.where(c == 0, DK0, DK1)

    def body(j, _):
        _unpack_chunk(ebuf, j, rowchunk, colchunk)
        pltpu.sync_copy(ones, acc.at[colchunk], add=True)
        return 0

    lax.fori_loop(0, nchunks, body, 0)
    plsc.subcore_barrier()
    pltpu.sync_copy(acc.at[pl.ds(base, STRIPE)], stage)
    pltpu.sync_copy(stage, out_hbm.at[pl.ds(c * ACC_ROWS + base, STRIPE)])


@functools.partial(
    pl.kernel,
    out_type=jax.ShapeDtypeStruct((NC, ACC_ROWS, D), jnp.float32),
    mesh=_mesh,
    scratch_types=[
        pltpu.VMEM((AK0, CH), jnp.int32),
        pltpu.VMEM((NBUF, CH), jnp.int32),
        pltpu.VMEM((NBUF, CH), jnp.int32),
        pltpu.VMEM((NBUF, CH, D), jnp.float32),
        pltpu.VMEM_SHARED((ACC_ROWS, D), jnp.float32),
        pltpu.SemaphoreType.DMA,
        pltpu.SemaphoreType.DMA,
    ],
)
def _agg_kernel(edges_hbm, y_hbm, out_hbm, ebuf, rowc, colc, mbuf, acc, sem0, sem1):
    sems = (sem0, sem1)
    c = lax.axis_index("c")
    s = lax.axis_index("s")
    _load_edges_split(edges_hbm, ebuf, c, s, AK0, AK1)
    _zero_vmem_2d(mbuf.at[0], CH, D)
    base = s * STRIPE
    for k in range(STRIPE // CH):
        pltpu.sync_copy(mbuf.at[0], acc.at[pl.ds(base + k * CH, CH)])
    pltpu.sync_copy(mbuf.at[0].at[pl.ds(0, STRIPE % CH)],
                    acc.at[pl.ds(base + (STRIPE // CH) * CH, STRIPE % CH)])
    plsc.subcore_barrier()
    nchunks = jnp.where(c == 0, AK0, AK1)

    for b in range(NBUF):
        _unpack_chunk(ebuf, b, rowc.at[b], colc.at[b])
        pltpu.async_copy(y_hbm.at[rowc.at[b]], mbuf.at[b], sems[b])

    def body(i, _):
        for b in range(NBUF):
            j = i * NBUF + b
            pltpu.make_async_copy(y_hbm.at[rowc.at[b]], mbuf.at[b], sems[b]).wait()
            pltpu.sync_copy(mbuf.at[b], acc.at[colc.at[b]], add=True)
            jn = jnp.minimum(j + NBUF, nchunks - 1)
            _unpack_chunk(ebuf, jn, rowc.at[b], colc.at[b])
            pltpu.async_copy(y_hbm.at[rowc.at[b]], mbuf.at[b], sems[b])
        return 0

    lax.fori_loop(0, nchunks // NBUF, body, 0)
    for b in range(NBUF):
        pltpu.make_async_copy(y_hbm.at[rowc.at[b]], mbuf.at[b], sems[b]).wait()
    plsc.subcore_barrier()
    pltpu.sync_copy(acc.at[pl.ds(base, STRIPE)], out_hbm.at[c, pl.ds(base, STRIPE)])


@functools.partial(
    pl.kernel,
    out_type=jax.ShapeDtypeStruct((NC * ACC_ROWS,), jnp.float32),
    mesh=_mesh,
    scratch_types=[
        pltpu.VMEM((DK0, CH), jnp.int32),
        pltpu.VMEM((NBUF, CH), jnp.int32),
        pltpu.VMEM((NBUF, CH), jnp.int32),
        pltpu.VMEM((NBUF, CH), jnp.float32),
        pltpu.VMEM((STRIPE,), jnp.float32),
        pltpu.VMEM_SHARED((ACC_ROWS,), jnp.float32),
        pltpu.SemaphoreType.DMA,
        pltpu.SemaphoreType.DMA,
    ],
)
def _agg1_kernel(edges_hbm, y_hbm, out_hbm, ebuf, rowc, colc, mbuf, stage, acc,
                 sem0, sem1):
    sems = (sem0, sem1)
    c = lax.axis_index("c")
    s = lax.axis_index("s")
    _load_edges_split(edges_hbm, ebuf, c, s, DK0, DK1)
    _zero_vmem_1d(mbuf.at[0], CH)
    base = s * STRIPE
    for k in range(STRIPE // CH):
        pltpu.sync_copy(mbuf.at[0], acc.at[pl.ds(base + k * CH, CH)])
    pltpu.sync_copy(mbuf.at[0].at[pl.ds(0, STRIPE % CH)],
                    acc.at[pl.ds(base + (STRIPE // CH) * CH, STRIPE % CH)])
    plsc.subcore_barrier()
    nchunks = jnp.where(c == 0, DK0, DK1)

    for b in range(NBUF):
        _unpack_chunk(ebuf, b, rowc.at[b], colc.at[b])
        pltpu.async_copy(y_hbm.at[rowc.at[b]], mbuf.at[b], sems[b])

    def body(i, _):
        for b in range(NBUF):
            j = i * NBUF + b
            pltpu.make_async_copy(y_hbm.at[rowc.at[b]], mbuf.at[b], sems[b]).wait()
            pltpu.sync_copy(mbuf.at[b], acc.at[colc.at[b]], add=True)
            jn = jnp.minimum(j + NBUF, nchunks - 1)
            _unpack_chunk(ebuf, jn, rowc.at[b], colc.at[b])
            pltpu.async_copy(y_hbm.at[rowc.at[b]], mbuf.at[b], sems[b])
        return 0

    lax.fori_loop(0, nchunks // NBUF, body, 0)
    for b in range(NBUF):
        pltpu.make_async_copy(y_hbm.at[rowc.at[b]], mbuf.at[b], sems[b]).wait()
    plsc.subcore_barrier()
    pltpu.sync_copy(acc.at[pl.ds(base, STRIPE)], stage)
    pltpu.sync_copy(stage, out_hbm.at[pl.ds(c * ACC_ROWS + base, STRIPE)])


RB = 1000
GRID = N // RB


def _stage_in_body(x_ref, w_ref, cnt0_ref, cnt1_ref, xw_ref, ya_ref, yb_ref,
                   dis2_ref, dis1_ref):
    xw = jnp.dot(x_ref[...], w_ref[...], preferred_element_type=jnp.float32)
    deg = cnt0_ref[...] + cnt1_ref[...]
    dis2 = lax.rsqrt(deg + 2.0)
    dis1 = lax.rsqrt(deg + 1.0)
    xw_ref[...] = xw
    y = dis2 * xw
    ya_ref[...] = y
    yb_ref[...] = y
    dis2_ref[...] = dis2
    dis1_ref[...] = dis1


def _stage_mid_body(agga_ref, aggb_ref, xw_ref, dis2_ref, b_ref, w_ref,
                    ypa_ref, ypb_ref, xw2_ref, y2a_ref, y2b_ref):
    del ypa_ref, ypb_ref
    dis2 = dis2_ref[...]
    agg = agga_ref[0] + agga_ref[1] + aggb_ref[0] + aggb_ref[1]
    h = dis2 * agg + dis2 * dis2 * xw_ref[...] + b_ref[...]
    h = jnp.maximum(h, 0.0)
    xw2 = jnp.dot(h, w_ref[...], preferred_element_type=jnp.float32)
    xw2_ref[...] = xw2
    y2 = dis2 * xw2
    y2a_ref[...] = y2
    y2b_ref[...] = y2


def _stage_out_body(agga_ref, aggb_ref, xw_ref, dis2_ref, dis1_ref, b_ref,
                    wo_ref, xwo_ref, yo_ref):
    dis2 = dis2_ref[...]
    agg = agga_ref[0] + agga_ref[1] + aggb_ref[0] + aggb_ref[1]
    h = dis2 * agg + dis2 * dis2 * xw_ref[...] + b_ref[...]
    h = jnp.maximum(h, 0.0)
    xwo = jnp.dot(h, wo_ref[...], preferred_element_type=jnp.float32)
    xwo_ref[...] = xwo
    yo_ref[...] = dis1_ref[...] * xwo


def _stage_final_body(aggo_ref, xwo_ref, dis1_ref, bo_ref, out_ref):
    dis1 = dis1_ref[...]
    out_ref[...] = dis1 * (aggo_ref[0] + aggo_ref[1]) + dis1 * dis1 * xwo_ref[...] + bo_ref[...]


def _rowspec(width):
    return pl.BlockSpec((RB, width), lambda i: (i, 0))


def _aggspec(width):
    return pl.BlockSpec((2, RB, width), lambda i: (0, i, 0))


def _fullspec(shape):
    return pl.BlockSpec(shape, lambda i: tuple(0 for _ in shape))


def kernel(x, edge_index, W1, b1, W2, b2, W3, b3, Wo, bo):
    row = edge_index[0]
    col = edge_index[1]
    pad = EP - E
    row_p = jnp.concatenate([row, jnp.zeros((pad,), jnp.int32)])
    col_p = jnp.concatenate([col, jnp.full((pad,), N, jnp.int32)])
    edges_p = (row_p | (col_p << 16)).reshape(NS * KTOT, CH)

    cnt = _deg_kernel(edges_p).reshape(NC, ACC_ROWS)
    cnt0 = cnt[0][:, None]
    cnt1 = cnt[1][:, None]

    stage_in = pl.pallas_call(
        _stage_in_body,
        grid=(GRID,),
        in_specs=[_rowspec(D), _fullspec((D, D)), _rowspec(1), _rowspec(1)],
        out_specs=[_rowspec(D), _rowspec(D), _rowspec(D), _rowspec(1), _rowspec(1)],
        out_shape=[
            jax.ShapeDtypeStruct((N, D), jnp.float32),
            jax.ShapeDtypeStruct((N, D), jnp.float32),
            jax.ShapeDtypeStruct((N, D), jnp.float32),
            jax.ShapeDtypeStruct((N, 1), jnp.float32),
            jax.ShapeDtypeStruct((N, 1), jnp.float32),
        ],
    )
    xw1, y1a, y1b, dis2, dis1 = stage_in(x, W1, cnt0, cnt1)

    stage_mid = pl.pallas_call(
        _stage_mid_body,
        grid=(GRID,),
        in_specs=[_aggspec(D), _aggspec(D), _rowspec(D), _rowspec(1),
                  _fullspec((1, D)), _fullspec((D, D)), _rowspec(D), _rowspec(D)],
        out_specs=[_rowspec(D), _rowspec(D), _rowspec(D)],
        out_shape=[
            jax.ShapeDtypeStruct((N, D), jnp.float32),
            jax.ShapeDtypeStruct((N, D), jnp.float32),
            jax.ShapeDtypeStruct((N, D), jnp.float32),
        ],
        input_output_aliases={6: 1, 7: 2},
    )

    e_a = edges_p[:NS * (KTOT // 2)]
    e_b = edges_p[NS * (KTOT // 2):]

    agg1a = _agg_kernel(e_a, y1a)
    agg1b = _agg_kernel(e_b, y1b)
    xw2, y2a, y2b = stage_mid(agg1a, agg1b, xw1, dis2, b1[None, :], W2, y1a, y1b)

    agg2a = _agg_kernel(e_a, y2a)
    agg2b = _agg_kernel(e_b, y2b)
    xw3, y3a, y3b = stage_mid(agg2a, agg2b, xw2, dis2, b2[None, :], W3, y2a, y2b)

    agg3a = _agg_kernel(e_a, y3a)
    agg3b = _agg_kernel(e_b, y3b)
    stage_out = pl.pallas_call(
        _stage_out_body,
        grid=(GRID,),
        in_specs=[_aggspec(D), _aggspec(D), _rowspec(D), _rowspec(1),
                  _rowspec(1), _fullspec((1, D)), _fullspec((D, 1))],
        out_specs=[_rowspec(1), _rowspec(1)],
        out_shape=[
            jax.ShapeDtypeStruct((N, 1), jnp.float32),
            jax.ShapeDtypeStruct((N, 1), jnp.float32),
        ],
    )
    xwo, yo = stage_out(agg3a, agg3b, xw3, dis2, dis1, b3[None, :], Wo)

    aggo = _agg1_kernel(edges_p, yo.reshape(N)).reshape(NC, ACC_ROWS)

    stage_final = pl.pallas_call(
        _stage_final_body,
        grid=(GRID,),
        in_specs=[_aggspec(1), _rowspec(1), _rowspec(1), _fullspec((1, 1))],
        out_specs=_rowspec(1),
        out_shape=jax.ShapeDtypeStruct((N, 1), jnp.float32),
    )
    out = stage_final(aggo[:, :, None], xwo, dis1, bo[None, :])
    return out
